# Initial kernel scaffold; baseline (speedup 1.0000x reference)
import jax
import jax.numpy as jnp
from jax.experimental import pallas as pl


def kernel(positive_triplets, negative_triplets, entities_emb, rel_embeddings):
    raise NotImplementedError("write your pallas kernel here")


# speedup vs baseline: 1.0183x; 1.0183x over previous
import functools

import jax
import jax.numpy as jnp
from jax import lax
from jax.experimental import pallas as pl
from jax.experimental.pallas import tpu as pltpu
from jax.experimental.pallas import tpu_sc as plsc

B = 16384
DIM = 64
MARGIN = 1.0
L = 16
NC = 2
NS = 16
NW = NC * NS


def _rsqrt(s):
    bits = lax.bitcast_convert_type(s, jnp.int32)
    y = lax.bitcast_convert_type(jnp.int32(0x5F3759DF) - (bits >> 1), jnp.float32)
    for _ in range(3):
        y = y * (1.5 - 0.5 * s * y * y)
    return y


def _build(BB, interpret=False):
    per_w = BB // NW
    cc = min(128, per_w)
    nchunk = per_w // cc
    ng = cc // L

    def distance16(hrows, rrows, trows, rows):
        zero = jnp.zeros((L,), jnp.float32)
        sh = [zero] * 4
        st = [zero] * 4
        for d in range(DIM):
            cd = jnp.full((L,), d, jnp.int32)
            hv = plsc.load_gather(hrows, [rows, cd])
            tv = plsc.load_gather(trows, [rows, cd])
            sh[d % 4] = sh[d % 4] + hv * hv
            st[d % 4] = st[d % 4] + tv * tv
        ih = _rsqrt(sh[0] + sh[1] + sh[2] + sh[3])
        it = _rsqrt(st[0] + st[1] + st[2] + st[3])
        acc = [zero] * 4
        for d in range(DIM):
            cd = jnp.full((L,), d, jnp.int32)
            hv = plsc.load_gather(hrows, [rows, cd])
            rv = plsc.load_gather(rrows, [rows, cd])
            tv = plsc.load_gather(trows, [rows, cd])
            acc[d % 4] = acc[d % 4] + jnp.abs(hv * ih + rv - tv * it)
        return acc[0] + acc[1] + acc[2] + acc[3]

    def body(ph, pr, pt, nh, nr, nt, ents, rels, out,
             phi, pri, pti, nhi, nri, nti,
             phr, prr, ptr, nhr, nrr, ntr,
             outv, sem):
        wid = lax.axis_index("s") * NC + lax.axis_index("c")
        iota = lax.iota(jnp.int32, L)

        def do_chunk(c, carry):
            base = wid * per_w + c * cc
            sl = pl.ds(base, cc)
            pltpu.sync_copy(ph.at[sl], phi)
            pltpu.sync_copy(pr.at[sl], pri)
            pltpu.sync_copy(pt.at[sl], pti)
            pltpu.sync_copy(nh.at[sl], nhi)
            pltpu.sync_copy(nr.at[sl], nri)
            pltpu.sync_copy(nt.at[sl], nti)
            cps = [
                pltpu.async_copy(ents.at[phi], phr, sem),
                pltpu.async_copy(rels.at[pri], prr, sem),
                pltpu.async_copy(ents.at[pti], ptr, sem),
                pltpu.async_copy(ents.at[nhi], nhr, sem),
                pltpu.async_copy(rels.at[nri], nrr, sem),
                pltpu.async_copy(ents.at[nti], ntr, sem),
            ]
            for cp in cps:
                cp.wait()

            def do_group(g, carry2):
                rows = g * L + iota
                pd = distance16(phr, prr, ptr, rows)
                nd = distance16(nhr, nrr, ntr, rows)
                outv[pl.ds(g * L, L)] = jnp.maximum(pd - nd + MARGIN, 0.0)
                return carry2

            lax.fori_loop(0, ng, do_group, 0)
            pltpu.sync_copy(outv, out.at[sl])
            return carry

        lax.fori_loop(0, nchunk, do_chunk, 0)

    return functools.partial(
        pl.kernel,
        out_type=jax.ShapeDtypeStruct((BB,), jnp.float32),
        mesh=plsc.VectorSubcoreMesh(
            core_axis_name="c", subcore_axis_name="s",
            num_cores=NC, num_subcores=NS),
        scratch_types=[
            pltpu.VMEM((cc,), jnp.int32),
            pltpu.VMEM((cc,), jnp.int32),
            pltpu.VMEM((cc,), jnp.int32),
            pltpu.VMEM((cc,), jnp.int32),
            pltpu.VMEM((cc,), jnp.int32),
            pltpu.VMEM((cc,), jnp.int32),
            pltpu.VMEM((cc, DIM), jnp.float32),
            pltpu.VMEM((cc, DIM), jnp.float32),
            pltpu.VMEM((cc, DIM), jnp.float32),
            pltpu.VMEM((cc, DIM), jnp.float32),
            pltpu.VMEM((cc, DIM), jnp.float32),
            pltpu.VMEM((cc, DIM), jnp.float32),
            pltpu.VMEM((cc,), jnp.float32),
            pltpu.SemaphoreType.DMA,
        ],
        compiler_params=pltpu.CompilerParams(
            use_tc_tiling_on_sc=False, needs_layout_passes=False),
        interpret=interpret,
    )(body)


_sc_call = _build(B)


def kernel(positive_triplets, negative_triplets, entities_emb, rel_embeddings):
    ph = positive_triplets[:, 0]
    pr = positive_triplets[:, 1]
    pt = positive_triplets[:, 2]
    nh = negative_triplets[:, 0]
    nr = negative_triplets[:, 1]
    nt = negative_triplets[:, 2]
    return _sc_call(ph, pr, pt, nh, nr, nt, entities_emb, rel_embeddings)
